# baseline (speedup 1.0000x reference)
import functools

import jax
import jax.numpy as jnp
import numpy as np
from jax.experimental import pallas as pl
from jax.experimental.pallas import tpu as pltpu

_BF = jnp.bfloat16
_VMEM = 64 * 1024 * 1024


def _cparams():
    return pltpu.CompilerParams(dimension_semantics=("parallel",),
                                vmem_limit_bytes=_VMEM)


def _conv3_body(x_ref, w_ref, G, H, W, C):
    x = x_ref[...]
    xc = jnp.concatenate([x[:, :, 0:W], x[:, :, 1:W + 1], x[:, :, 2:W + 2]],
                         axis=-1)
    acc = jnp.dot(xc[:, 0:H].reshape(G * H * W, 3 * C), w_ref[0],
                  preferred_element_type=jnp.float32)
    acc = acc + jnp.dot(xc[:, 1:H + 1].reshape(G * H * W, 3 * C), w_ref[1],
                        preferred_element_type=jnp.float32)
    acc = acc + jnp.dot(xc[:, 2:H + 2].reshape(G * H * W, 3 * C), w_ref[2],
                        preferred_element_type=jnp.float32)
    return acc


def _sconv3_body(s_ref, w_ref, G, H, W, C):
    return _conv3_body(s_ref, w_ref, G, H, W, C)


def _id_block(x, s_ref, w1_ref, b1_ref, w2_ref, b2_ref, G, H, W, C):
    s_ref[...] = jnp.zeros_like(s_ref)
    s_ref[:, 1:H + 1, 1:W + 1, :] = x
    acc1 = _sconv3_body(s_ref, w1_ref, G, H, W, C) + b1_ref[...]
    y = jnp.maximum(acc1, 0.0).astype(_BF).reshape(G, H, W, C)
    s_ref[:, 1:H + 1, 1:W + 1, :] = y
    acc2 = _sconv3_body(s_ref, w2_ref, G, H, W, C) + b2_ref[...]
    acc2 = acc2 + x.reshape(G * H * W, C).astype(jnp.float32)
    return jnp.maximum(acc2, 0.0).astype(_BF).reshape(G, H, W, C)


def _stage_kernel(x_ref, w1_ref, b1_ref, w2_ref, b2_ref, w3_ref, b3_ref,
                  w4_ref, b4_ref, o_ref, s_ref, *, G, H, W, C):
    h = _id_block(x_ref[...], s_ref, w1_ref, b1_ref, w2_ref, b2_ref,
                  G, H, W, C)
    h = _id_block(h, s_ref, w3_ref, b3_ref, w4_ref, b4_ref, G, H, W, C)
    o_ref[...] = h.astype(o_ref.dtype)


def _dstage_kernel(x_ref, w1_ref, b1_ref, w2_ref, b2_ref, wsc_ref, bsc_ref,
                   w3_ref, b3_ref, w4_ref, b4_ref, o_ref, s_ref,
                   *, G, Ho, C):
    xs = x_ref[...]
    pieces = [xs[:, a:a + Ho, b:b + Ho, :]
              for a in range(2) for b in range(2)]
    p = jnp.concatenate(pieces, axis=-1).reshape(G * Ho * Ho, 16 * C)
    acc1 = jnp.dot(p, w1_ref[...], preferred_element_type=jnp.float32)
    y = jnp.maximum(acc1 + b1_ref[...], 0.0).astype(_BF)
    N = y.shape[-1]
    s_ref[...] = jnp.zeros_like(s_ref)
    s_ref[:, 1:Ho + 1, 1:Ho + 1, :] = y.reshape(G, Ho, Ho, N)
    acc2 = _sconv3_body(s_ref, w2_ref, G, Ho, Ho, N) + b2_ref[...]
    x00 = xs[:, 0:Ho, 0:Ho, 3 * C:4 * C].reshape(G * Ho * Ho, C)
    sc = jnp.dot(x00, wsc_ref[...], preferred_element_type=jnp.float32)
    acc2 = acc2 + sc + bsc_ref[...]
    h = jnp.maximum(acc2, 0.0).astype(_BF).reshape(G, Ho, Ho, N)
    h = _id_block(h, s_ref, w3_ref, b3_ref, w4_ref, b4_ref, G, Ho, Ho, N)
    o_ref[...] = h.astype(o_ref.dtype)


def _gate_kernel(x_ref, w_ref, b_ref, o_ref, *, Ho, Co):
    xs = x_ref[0]
    pieces = [xs[gr:gr + Ho, gc:gc + Ho, :]
              for gr in range(3) for gc in range(3)]
    p = jnp.concatenate(pieces, axis=-1).reshape(Ho * Ho, 432)

    def phase(k):
        y = jnp.dot(p, w_ref[k], preferred_element_type=jnp.float32)
        return jnp.maximum(y + b_ref[...], 0.0).reshape(Ho, Ho, Co)

    yee, yeo, yoe, yoo = phase(0), phase(1), phase(2), phase(3)
    zr = jnp.zeros((Ho, 1, Co), jnp.float32)
    zd = jnp.zeros((1, Ho, Co), jnp.float32)

    def sr(a):
        return jnp.concatenate([zr, a[:, :-1]], axis=1)

    def sd(a):
        return jnp.concatenate([zd, a[:-1]], axis=0)

    m = jnp.maximum(yee, jnp.maximum(yeo, sr(yeo)))
    m = jnp.maximum(m, jnp.maximum(yoe, sd(yoe)))
    oo = jnp.maximum(jnp.maximum(yoo, sd(yoo)),
                     jnp.maximum(sr(yoo), sd(sr(yoo))))
    m = jnp.maximum(m, oo)
    o_ref[...] = m[None].astype(o_ref.dtype)


def _wspec(shape):
    n = len(shape)
    return pl.BlockSpec(shape, lambda i: (0,) * n)


def _stage(x, w1, b1, w2, b2, w3, b3, w4, b4, *, G=1, out_dtype=_BF):
    B, H, W, C = x.shape
    G = min(G, B)
    wsp = [_wspec((3, 3 * C, C)), _wspec((1, C))] * 4
    return pl.pallas_call(
        functools.partial(_stage_kernel, G=G, H=H, W=W, C=C),
        out_shape=jax.ShapeDtypeStruct((B, H, W, C), out_dtype),
        grid=(B // G,),
        in_specs=[pl.BlockSpec((G, H, W, C), lambda i: (i, 0, 0, 0))] + wsp,
        out_specs=pl.BlockSpec((G, H, W, C), lambda i: (i, 0, 0, 0)),
        scratch_shapes=[pltpu.VMEM((G, H + 2, W + 2, C), _BF)],
        compiler_params=_cparams(),
    )(x, w1, b1, w2, b2, w3, b3, w4, b4)


def _dstage(xs, w16, b1, w2, b2, wsc, bsc, w3, b3, w4, b4,
            *, Ho, C, N, G=1, out_dtype=_BF):
    B, Hs, _, _ = xs.shape
    G = min(G, B)
    return pl.pallas_call(
        functools.partial(_dstage_kernel, G=G, Ho=Ho, C=C),
        out_shape=jax.ShapeDtypeStruct((B, Ho, Ho, N), out_dtype),
        grid=(B // G,),
        in_specs=[
            pl.BlockSpec((G, Hs, Hs, 4 * C), lambda i: (i, 0, 0, 0)),
            _wspec((16 * C, N)), _wspec((1, N)),
            _wspec((3, 3 * N, N)), _wspec((1, N)),
            _wspec((C, N)), _wspec((1, N)),
            _wspec((3, 3 * N, N)), _wspec((1, N)),
            _wspec((3, 3 * N, N)), _wspec((1, N)),
        ],
        out_specs=pl.BlockSpec((G, Ho, Ho, N), lambda i: (i, 0, 0, 0)),
        scratch_shapes=[pltpu.VMEM((G, Ho + 2, Ho + 2, N), _BF)],
        compiler_params=_cparams(),
    )(xs, w16, b1, w2, b2, wsc, bsc, w3, b3, w4, b4)


def _gate(xs4, w4, bg):
    B = xs4.shape[0]
    return pl.pallas_call(
        functools.partial(_gate_kernel, Ho=56, Co=64),
        out_shape=jax.ShapeDtypeStruct((B, 56, 56, 64), _BF),
        grid=(B,),
        in_specs=[
            pl.BlockSpec((1, 60, 60, 48), lambda i: (i, 0, 0, 0)),
            pl.BlockSpec((4, 432, 64), lambda i: (0, 0, 0)),
            pl.BlockSpec((1, 64), lambda i: (0, 0)),
        ],
        out_specs=pl.BlockSpec((1, 56, 56, 64), lambda i: (i, 0, 0, 0)),
        compiler_params=_cparams(),
    )(xs4, w4, bg)


def _fold3(w, scale):
    c_in, c_out = w.shape[2], w.shape[3]
    return (w.reshape(3, 3 * c_in, c_out) * scale[None, None, :]).astype(_BF)


def _bias(shift):
    return shift.reshape(1, -1).astype(jnp.float32)


def _s2d2(x):
    B, H, W, C = x.shape
    xp = jnp.pad(x, ((0, 0), (1, 3), (1, 3), (0, 0)))
    Hs = (H + 4) // 2
    return xp.reshape(B, Hs, 2, Hs, 2, C).transpose(0, 1, 3, 2, 4, 5).reshape(
        B, Hs, Hs, 4 * C)


_IDX2 = np.array([[0, 1], [2, 3]], dtype=np.int32)


def _pack_w16(w, scale):
    cin, cout = w.shape[2], w.shape[3]
    wf = w * scale[None, None, None, :]
    w4 = jnp.pad(wf, ((0, 1), (0, 1), (0, 0), (0, 0)))
    idx = jnp.asarray(_IDX2)
    wr = w4[idx]
    wrc = wr[:, :, idx]
    return wrc.transpose(0, 2, 1, 3, 4, 5).reshape(16 * cin, cout).astype(_BF)


def _pack_gate_w(gate_w, gate_scale):
    wf = gate_w * gate_scale[None, None, None, :]
    w8 = jnp.pad(wf, ((0, 1), (0, 1), (0, 0), (0, 0)))
    idx = np.full((2, 3, 4), 7, dtype=np.int32)
    for a in range(2):
        for g in range(3):
            for d in range(4):
                kh = 4 * g + d - 2 * a
                if 0 <= kh <= 6:
                    idx[a, g, d] = kh
    idx = jnp.asarray(idx)
    wr = w8[idx]
    wrc = wr[:, :, :, idx]
    w4 = wrc.transpose(0, 3, 1, 4, 6, 2, 5, 7).reshape(4, 432, 64)
    return w4.astype(_BF)


def kernel(
    x,
    gate_w, gate_scale, gate_shift,
    blk0_conv1_w, blk0_conv1_scale, blk0_conv1_shift,
    blk0_conv2_w, blk0_conv2_scale, blk0_conv2_shift,
    blk1_conv1_w, blk1_conv1_scale, blk1_conv1_shift,
    blk1_conv2_w, blk1_conv2_scale, blk1_conv2_shift,
    blk2_conv1_w, blk2_conv1_scale, blk2_conv1_shift,
    blk2_conv2_w, blk2_conv2_scale, blk2_conv2_shift,
    blk2_sc_w, blk2_sc_scale, blk2_sc_shift,
    blk3_conv1_w, blk3_conv1_scale, blk3_conv1_shift,
    blk3_conv2_w, blk3_conv2_scale, blk3_conv2_shift,
    blk4_conv1_w, blk4_conv1_scale, blk4_conv1_shift,
    blk4_conv2_w, blk4_conv2_scale, blk4_conv2_shift,
    blk4_sc_w, blk4_sc_scale, blk4_sc_shift,
    blk5_conv1_w, blk5_conv1_scale, blk5_conv1_shift,
    blk5_conv2_w, blk5_conv2_scale, blk5_conv2_shift,
    blk6_conv1_w, blk6_conv1_scale, blk6_conv1_shift,
    blk6_conv2_w, blk6_conv2_scale, blk6_conv2_shift,
    blk6_sc_w, blk6_sc_scale, blk6_sc_shift,
    blk7_conv1_w, blk7_conv1_scale, blk7_conv1_shift,
    blk7_conv2_w, blk7_conv2_scale, blk7_conv2_shift,
):
    B = x.shape[0]

    xb = jnp.pad(x.astype(_BF), ((0, 0), (0, 0), (3, 13), (3, 13)))
    xs4 = xb.reshape(B, 3, 60, 4, 60, 4).transpose(0, 2, 4, 1, 3, 5).reshape(
        B, 60, 60, 48)
    g = _gate(xs4, _pack_gate_w(gate_w, gate_scale), _bias(gate_shift))

    h = _stage(g,
               _fold3(blk0_conv1_w, blk0_conv1_scale), _bias(blk0_conv1_shift),
               _fold3(blk0_conv2_w, blk0_conv2_scale), _bias(blk0_conv2_shift),
               _fold3(blk1_conv1_w, blk1_conv1_scale), _bias(blk1_conv1_shift),
               _fold3(blk1_conv2_w, blk1_conv2_scale), _bias(blk1_conv2_shift),
               G=4)

    def dstage(h_in, Ho, C, N, w1, s1, sh1, w2, s2, sh2, wsc, ssc, shsc,
               w3, s3, sh3, w4, s4, sh4, G, out_dtype=_BF):
        return _dstage(_s2d2(h_in), _pack_w16(w1, s1), _bias(sh1),
                       _fold3(w2, s2), _bias(sh2),
                       (wsc.reshape(C, N) * ssc[None, :]).astype(_BF),
                       _bias(shsc),
                       _fold3(w3, s3), _bias(sh3),
                       _fold3(w4, s4), _bias(sh4),
                       Ho=Ho, C=C, N=N, G=G, out_dtype=out_dtype)

    h = dstage(h, 28, 64, 128,
               blk2_conv1_w, blk2_conv1_scale, blk2_conv1_shift,
               blk2_conv2_w, blk2_conv2_scale, blk2_conv2_shift,
               blk2_sc_w, blk2_sc_scale, blk2_sc_shift,
               blk3_conv1_w, blk3_conv1_scale, blk3_conv1_shift,
               blk3_conv2_w, blk3_conv2_scale, blk3_conv2_shift, G=4)
    h = dstage(h, 14, 128, 256,
               blk4_conv1_w, blk4_conv1_scale, blk4_conv1_shift,
               blk4_conv2_w, blk4_conv2_scale, blk4_conv2_shift,
               blk4_sc_w, blk4_sc_scale, blk4_sc_shift,
               blk5_conv1_w, blk5_conv1_scale, blk5_conv1_shift,
               blk5_conv2_w, blk5_conv2_scale, blk5_conv2_shift, G=8)
    h = dstage(h, 7, 256, 512,
               blk6_conv1_w, blk6_conv1_scale, blk6_conv1_shift,
               blk6_conv2_w, blk6_conv2_scale, blk6_conv2_shift,
               blk6_sc_w, blk6_sc_scale, blk6_sc_shift,
               blk7_conv1_w, blk7_conv1_scale, blk7_conv1_shift,
               blk7_conv2_w, blk7_conv2_scale, blk7_conv2_shift, G=16,
               out_dtype=jnp.float32)

    return jnp.transpose(h, (0, 3, 1, 2))


# speedup vs baseline: 1.0085x; 1.0085x over previous
import functools

import jax
import jax.numpy as jnp
import numpy as np
from jax.experimental import pallas as pl
from jax.experimental.pallas import tpu as pltpu

_BF = jnp.bfloat16
_VMEM = 64 * 1024 * 1024


def _cparams():
    return pltpu.CompilerParams(dimension_semantics=("parallel",),
                                vmem_limit_bytes=_VMEM)


def _conv3_body(x_ref, w_ref, G, H, W, C):
    x = x_ref[...]
    xc = jnp.concatenate([x[:, :, 0:W], x[:, :, 1:W + 1], x[:, :, 2:W + 2]],
                         axis=-1)
    acc = jnp.dot(xc[:, 0:H].reshape(G * H * W, 3 * C), w_ref[0],
                  preferred_element_type=jnp.float32)
    acc = acc + jnp.dot(xc[:, 1:H + 1].reshape(G * H * W, 3 * C), w_ref[1],
                        preferred_element_type=jnp.float32)
    acc = acc + jnp.dot(xc[:, 2:H + 2].reshape(G * H * W, 3 * C), w_ref[2],
                        preferred_element_type=jnp.float32)
    return acc


def _sconv3_body(s_ref, w_ref, G, H, W, C):
    return _conv3_body(s_ref, w_ref, G, H, W, C)


def _id_block(x, s_ref, w1_ref, b1_ref, w2_ref, b2_ref, G, H, W, C):
    s_ref[...] = jnp.zeros_like(s_ref)
    s_ref[:, 1:H + 1, 1:W + 1, :] = x
    acc1 = _sconv3_body(s_ref, w1_ref, G, H, W, C) + b1_ref[...]
    y = jnp.maximum(acc1, 0.0).astype(_BF).reshape(G, H, W, C)
    s_ref[:, 1:H + 1, 1:W + 1, :] = y
    acc2 = _sconv3_body(s_ref, w2_ref, G, H, W, C) + b2_ref[...]
    acc2 = acc2 + x.reshape(G * H * W, C).astype(jnp.float32)
    return jnp.maximum(acc2, 0.0).astype(_BF).reshape(G, H, W, C)


def _stage_kernel(x_ref, w1_ref, b1_ref, w2_ref, b2_ref, w3_ref, b3_ref,
                  w4_ref, b4_ref, o_ref, s_ref, *, G, H, W, C):
    h = _id_block(x_ref[...], s_ref, w1_ref, b1_ref, w2_ref, b2_ref,
                  G, H, W, C)
    h = _id_block(h, s_ref, w3_ref, b3_ref, w4_ref, b4_ref, G, H, W, C)
    o_ref[...] = h.astype(o_ref.dtype)


def _dstage_kernel(x_ref, w1_ref, b1_ref, w2_ref, b2_ref, wsc_ref, bsc_ref,
                   w3_ref, b3_ref, w4_ref, b4_ref, o_ref, s_ref,
                   *, G, Ho, C):
    xs = x_ref[...]
    pieces = [xs[:, a:a + Ho, b:b + Ho, :]
              for a in range(2) for b in range(2)]
    p = jnp.concatenate(pieces, axis=-1).reshape(G * Ho * Ho, 16 * C)
    acc1 = jnp.dot(p, w1_ref[...], preferred_element_type=jnp.float32)
    y = jnp.maximum(acc1 + b1_ref[...], 0.0).astype(_BF)
    N = y.shape[-1]
    s_ref[...] = jnp.zeros_like(s_ref)
    s_ref[:, 1:Ho + 1, 1:Ho + 1, :] = y.reshape(G, Ho, Ho, N)
    acc2 = _sconv3_body(s_ref, w2_ref, G, Ho, Ho, N) + b2_ref[...]
    x00 = xs[:, 0:Ho, 0:Ho, 3 * C:4 * C].reshape(G * Ho * Ho, C)
    sc = jnp.dot(x00, wsc_ref[...], preferred_element_type=jnp.float32)
    acc2 = acc2 + sc + bsc_ref[...]
    h = jnp.maximum(acc2, 0.0).astype(_BF).reshape(G, Ho, Ho, N)
    h = _id_block(h, s_ref, w3_ref, b3_ref, w4_ref, b4_ref, G, Ho, Ho, N)
    o_ref[...] = h.astype(o_ref.dtype)


def _gate_kernel(x_ref, w_ref, b_ref, o_ref, *, Ho, Co):
    xs = x_ref[0]
    pieces = [xs[gr:gr + Ho, gc:gc + Ho, :]
              for gr in range(3) for gc in range(3)]
    p = jnp.concatenate(pieces, axis=-1).reshape(Ho * Ho, 432)

    def phase(k):
        y = jnp.dot(p, w_ref[k], preferred_element_type=jnp.float32)
        return jnp.maximum(y + b_ref[...], 0.0).reshape(Ho, Ho, Co)

    yee, yeo, yoe, yoo = phase(0), phase(1), phase(2), phase(3)
    zr = jnp.zeros((Ho, 1, Co), jnp.float32)
    zd = jnp.zeros((1, Ho, Co), jnp.float32)

    def sr(a):
        return jnp.concatenate([zr, a[:, :-1]], axis=1)

    def sd(a):
        return jnp.concatenate([zd, a[:-1]], axis=0)

    m = jnp.maximum(yee, jnp.maximum(yeo, sr(yeo)))
    m = jnp.maximum(m, jnp.maximum(yoe, sd(yoe)))
    oo = jnp.maximum(jnp.maximum(yoo, sd(yoo)),
                     jnp.maximum(sr(yoo), sd(sr(yoo))))
    m = jnp.maximum(m, oo)
    o_ref[...] = m[None].astype(o_ref.dtype)


def _wspec(shape):
    n = len(shape)
    return pl.BlockSpec(shape, lambda i: (0,) * n)


def _stage(x, w1, b1, w2, b2, w3, b3, w4, b4, *, G=1, out_dtype=_BF):
    B, H, W, C = x.shape
    G = min(G, B)
    wsp = [_wspec((3, 3 * C, C)), _wspec((1, C))] * 4
    return pl.pallas_call(
        functools.partial(_stage_kernel, G=G, H=H, W=W, C=C),
        out_shape=jax.ShapeDtypeStruct((B, H, W, C), out_dtype),
        grid=(B // G,),
        in_specs=[pl.BlockSpec((G, H, W, C), lambda i: (i, 0, 0, 0))] + wsp,
        out_specs=pl.BlockSpec((G, H, W, C), lambda i: (i, 0, 0, 0)),
        scratch_shapes=[pltpu.VMEM((G, H + 2, W + 2, C), _BF)],
        compiler_params=_cparams(),
    )(x, w1, b1, w2, b2, w3, b3, w4, b4)


def _dstage(xs, w16, b1, w2, b2, wsc, bsc, w3, b3, w4, b4,
            *, Ho, C, N, G=1, out_dtype=_BF):
    B, Hs, _, _ = xs.shape
    G = min(G, B)
    return pl.pallas_call(
        functools.partial(_dstage_kernel, G=G, Ho=Ho, C=C),
        out_shape=jax.ShapeDtypeStruct((B, Ho, Ho, N), out_dtype),
        grid=(B // G,),
        in_specs=[
            pl.BlockSpec((G, Hs, Hs, 4 * C), lambda i: (i, 0, 0, 0)),
            _wspec((16 * C, N)), _wspec((1, N)),
            _wspec((3, 3 * N, N)), _wspec((1, N)),
            _wspec((C, N)), _wspec((1, N)),
            _wspec((3, 3 * N, N)), _wspec((1, N)),
            _wspec((3, 3 * N, N)), _wspec((1, N)),
        ],
        out_specs=pl.BlockSpec((G, Ho, Ho, N), lambda i: (i, 0, 0, 0)),
        scratch_shapes=[pltpu.VMEM((G, Ho + 2, Ho + 2, N), _BF)],
        compiler_params=_cparams(),
    )(xs, w16, b1, w2, b2, wsc, bsc, w3, b3, w4, b4)


def _gate(xs4, w4, bg):
    B = xs4.shape[0]
    return pl.pallas_call(
        functools.partial(_gate_kernel, Ho=56, Co=64),
        out_shape=jax.ShapeDtypeStruct((B, 56, 56, 64), _BF),
        grid=(B,),
        in_specs=[
            pl.BlockSpec((1, 60, 60, 48), lambda i: (i, 0, 0, 0)),
            pl.BlockSpec((4, 432, 64), lambda i: (0, 0, 0)),
            pl.BlockSpec((1, 64), lambda i: (0, 0)),
        ],
        out_specs=pl.BlockSpec((1, 56, 56, 64), lambda i: (i, 0, 0, 0)),
        compiler_params=_cparams(),
    )(xs4, w4, bg)


def _fold3(w, scale):
    c_in, c_out = w.shape[2], w.shape[3]
    return (w.reshape(3, 3 * c_in, c_out) * scale[None, None, :]).astype(_BF)


def _bias(shift):
    return shift.reshape(1, -1).astype(jnp.float32)


def _s2d2(x):
    B, H, W, C = x.shape
    xp = jnp.pad(x, ((0, 0), (1, 3), (1, 3), (0, 0)))
    Hs = (H + 4) // 2
    return xp.reshape(B, Hs, 2, Hs, 2, C).transpose(0, 1, 3, 2, 4, 5).reshape(
        B, Hs, Hs, 4 * C)


_IDX2 = np.array([[0, 1], [2, 3]], dtype=np.int32)


def _pack_w16(w, scale):
    cin, cout = w.shape[2], w.shape[3]
    wf = w * scale[None, None, None, :]
    w4 = jnp.pad(wf, ((0, 1), (0, 1), (0, 0), (0, 0)))
    idx = jnp.asarray(_IDX2)
    wr = w4[idx]
    wrc = wr[:, :, idx]
    return wrc.transpose(0, 2, 1, 3, 4, 5).reshape(16 * cin, cout).astype(_BF)


def _pack_gate_w(gate_w, gate_scale):
    wf = gate_w * gate_scale[None, None, None, :]
    w8 = jnp.pad(wf, ((0, 1), (0, 1), (0, 0), (0, 0)))
    idx = np.full((2, 3, 4), 7, dtype=np.int32)
    for a in range(2):
        for g in range(3):
            for d in range(4):
                kh = 4 * g + d - 2 * a
                if 0 <= kh <= 6:
                    idx[a, g, d] = kh
    idx = jnp.asarray(idx)
    wr = w8[idx]
    wrc = wr[:, :, :, idx]
    w4 = wrc.transpose(0, 3, 1, 4, 6, 2, 5, 7).reshape(4, 432, 64)
    return w4.astype(_BF)


def kernel(
    x,
    gate_w, gate_scale, gate_shift,
    blk0_conv1_w, blk0_conv1_scale, blk0_conv1_shift,
    blk0_conv2_w, blk0_conv2_scale, blk0_conv2_shift,
    blk1_conv1_w, blk1_conv1_scale, blk1_conv1_shift,
    blk1_conv2_w, blk1_conv2_scale, blk1_conv2_shift,
    blk2_conv1_w, blk2_conv1_scale, blk2_conv1_shift,
    blk2_conv2_w, blk2_conv2_scale, blk2_conv2_shift,
    blk2_sc_w, blk2_sc_scale, blk2_sc_shift,
    blk3_conv1_w, blk3_conv1_scale, blk3_conv1_shift,
    blk3_conv2_w, blk3_conv2_scale, blk3_conv2_shift,
    blk4_conv1_w, blk4_conv1_scale, blk4_conv1_shift,
    blk4_conv2_w, blk4_conv2_scale, blk4_conv2_shift,
    blk4_sc_w, blk4_sc_scale, blk4_sc_shift,
    blk5_conv1_w, blk5_conv1_scale, blk5_conv1_shift,
    blk5_conv2_w, blk5_conv2_scale, blk5_conv2_shift,
    blk6_conv1_w, blk6_conv1_scale, blk6_conv1_shift,
    blk6_conv2_w, blk6_conv2_scale, blk6_conv2_shift,
    blk6_sc_w, blk6_sc_scale, blk6_sc_shift,
    blk7_conv1_w, blk7_conv1_scale, blk7_conv1_shift,
    blk7_conv2_w, blk7_conv2_scale, blk7_conv2_shift,
):
    B = x.shape[0]

    xb = jnp.pad(x.astype(_BF), ((0, 0), (0, 0), (3, 13), (3, 13)))
    xs4 = xb.reshape(B, 3, 60, 4, 60, 4).transpose(0, 2, 4, 1, 3, 5).reshape(
        B, 60, 60, 48)
    g = _gate(xs4, _pack_gate_w(gate_w, gate_scale), _bias(gate_shift))

    h = _stage(g,
               _fold3(blk0_conv1_w, blk0_conv1_scale), _bias(blk0_conv1_shift),
               _fold3(blk0_conv2_w, blk0_conv2_scale), _bias(blk0_conv2_shift),
               _fold3(blk1_conv1_w, blk1_conv1_scale), _bias(blk1_conv1_shift),
               _fold3(blk1_conv2_w, blk1_conv2_scale), _bias(blk1_conv2_shift),
               G=2)

    def dstage(h_in, Ho, C, N, w1, s1, sh1, w2, s2, sh2, wsc, ssc, shsc,
               w3, s3, sh3, w4, s4, sh4, G, out_dtype=_BF):
        return _dstage(_s2d2(h_in), _pack_w16(w1, s1), _bias(sh1),
                       _fold3(w2, s2), _bias(sh2),
                       (wsc.reshape(C, N) * ssc[None, :]).astype(_BF),
                       _bias(shsc),
                       _fold3(w3, s3), _bias(sh3),
                       _fold3(w4, s4), _bias(sh4),
                       Ho=Ho, C=C, N=N, G=G, out_dtype=out_dtype)

    h = dstage(h, 28, 64, 128,
               blk2_conv1_w, blk2_conv1_scale, blk2_conv1_shift,
               blk2_conv2_w, blk2_conv2_scale, blk2_conv2_shift,
               blk2_sc_w, blk2_sc_scale, blk2_sc_shift,
               blk3_conv1_w, blk3_conv1_scale, blk3_conv1_shift,
               blk3_conv2_w, blk3_conv2_scale, blk3_conv2_shift, G=2)
    h = dstage(h, 14, 128, 256,
               blk4_conv1_w, blk4_conv1_scale, blk4_conv1_shift,
               blk4_conv2_w, blk4_conv2_scale, blk4_conv2_shift,
               blk4_sc_w, blk4_sc_scale, blk4_sc_shift,
               blk5_conv1_w, blk5_conv1_scale, blk5_conv1_shift,
               blk5_conv2_w, blk5_conv2_scale, blk5_conv2_shift, G=8)
    h = dstage(h, 7, 256, 512,
               blk6_conv1_w, blk6_conv1_scale, blk6_conv1_shift,
               blk6_conv2_w, blk6_conv2_scale, blk6_conv2_shift,
               blk6_sc_w, blk6_sc_scale, blk6_sc_shift,
               blk7_conv1_w, blk7_conv1_scale, blk7_conv1_shift,
               blk7_conv2_w, blk7_conv2_scale, blk7_conv2_shift, G=16,
               out_dtype=jnp.float32)

    return jnp.transpose(h, (0, 3, 1, 2))
